# Initial kernel scaffold; baseline (speedup 1.0000x reference)
import jax
import jax.numpy as jnp
from jax.experimental import pallas as pl


def kernel(tokens, idx, grid_size):
    raise NotImplementedError("write your pallas kernel here")


# speedup vs baseline: 11.4683x; 11.4683x over previous
import functools

import jax
import jax.numpy as jnp
from jax import lax
from jax.experimental import pallas as pl
from jax.experimental.pallas import tpu as pltpu
from jax.experimental.pallas import tpu_sc as plsc

B, N_TOK, D = 8, 24576, 128
GRID = 49152
NC, NS, L = 2, 16, 16
NW = NC * NS
QPB = NW // B
RANGE = GRID // QPB
CHUNK = 128
NBLK = RANGE // CHUNK
ZGRP = 8
WIN = 2048
NWIN = N_TOK // WIN
VPW = WIN // L
SENT = 0x7FFFFFFF


def _winner_scatter(tokens, idx):
  mesh = plsc.VectorSubcoreMesh(
      core_axis_name="c", subcore_axis_name="s",
      num_cores=NC, num_subcores=NS)

  @functools.partial(
      pl.kernel,
      out_type=jax.ShapeDtypeStruct((B, GRID, D), jnp.float32),
      mesh=mesh,
      compiler_params=pltpu.CompilerParams(needs_layout_passes=False),
      scratch_types=[
          pltpu.VMEM((2, WIN), jnp.int32),
          pltpu.VMEM((RANGE,), jnp.int32),
          pltpu.VMEM((L,), jnp.int32),
          pltpu.VMEM((RANGE + L,), jnp.int32),
          pltpu.VMEM((RANGE + L,), jnp.int32),
          pltpu.VMEM((NBLK, CHUNK), jnp.int32),
          pltpu.VMEM((CHUNK, D), jnp.float32),
          pltpu.VMEM((2, CHUNK, D), jnp.float32),
          pltpu.SemaphoreType.DMA,
          pltpu.SemaphoreType.DMA,
          pltpu.SemaphoreType.DMA,
          pltpu.SemaphoreType.DMA,
      ],
  )
  def body(tokens_hbm, idx_hbm, out_hbm, idx_win, inv, bounce,
           wtok, wcell, wcell2, zblk, rows, sem_i, sem_z, sem_g, sem_s):
    wid = lax.axis_index("s") * NC + lax.axis_index("c")
    b = wid // QPB
    base = (wid % QPB) * RANGE

    iota = lax.iota(jnp.int32, L)
    zeros16f = jnp.zeros((L,), jnp.float32)
    neg16 = jnp.full((L,), -1, jnp.int32)
    shift_idx = jnp.minimum(iota + 1, L - 1)

    def init_inv(j, _):
      inv[pl.ds(j * L, L)] = neg16
      return 0
    lax.fori_loop(0, RANGE // L, init_inv, 0)

    def init_z(i, _):
      r = i // (D // L)
      c = (i % (D // L)) * L
      zblk[r, pl.ds(c, L)] = zeros16f
      return 0
    lax.fori_loop(0, CHUNK * (D // L), init_z, 0)

    out_b = out_hbm.at[b]
    tok_b = tokens_hbm.at[b]

    def zero_start(g):
      for t in range(ZGRP):
        blk = g * ZGRP + t
        pltpu.async_copy(
            zblk, out_b.at[pl.ds(base + blk * CHUNK, CHUNK)], sem_z)

    def zero_drain(g):
      for t in range(ZGRP):
        blk = g * ZGRP + t
        pltpu.make_async_copy(
            zblk, out_b.at[pl.ds(base + blk * CHUNK, CHUNK)], sem_z).wait()

    pltpu.async_copy(idx_hbm.at[b, pl.ds(0, WIN)], idx_win.at[0], sem_i)
    for w in range(NWIN):
      if w + 1 < NWIN:
        pltpu.async_copy(idx_hbm.at[b, pl.ds((w + 1) * WIN, WIN)],
                         idx_win.at[(w + 1) % 2], sem_i)
      pltpu.make_async_copy(idx_hbm.at[b, pl.ds(w * WIN, WIN)],
                            idx_win.at[w % 2], sem_i).wait()
      zero_start(w)

      def vreg_body(k, _, w=w):
        v = idx_win[w % 2, pl.ds(k * L, L)]
        local = v - base
        m = (local >= 0) & (local < RANGE)
        p = (w * WIN + k * L) + iota
        key = jnp.where(m, (local << 15) | p, SENT)
        skey, _ = plsc.sort_key_val(key, key)
        bounce[...] = skey
        snext = plsc.load_gather(bounce, [shift_idx])
        keep = ((skey >> 15) != (snext >> 15)) | (iota == L - 1)
        valid = skey != SENT
        plsc.store_scatter(inv, [skey >> 15], skey & 0x7FFF,
                           mask=keep & valid)
        return 0
      lax.fori_loop(0, VPW, vreg_body, 0)
      if w >= 1:
        zero_drain(w - 1)
    zero_drain(NWIN - 1)

    def extract(j, cnt):
      v = inv[pl.ds(j * L, L)]
      m = v >= 0
      plsc.store_compressed(wtok.at[pl.ds(cnt, L)], v, mask=m)
      plsc.store_compressed(wcell.at[pl.ds(cnt, L)],
                            base + j * L + iota, mask=m)
      return cnt + jnp.max(plsc.all_reduce_population_count(m))
    cnt = lax.fori_loop(0, RANGE // L, extract, jnp.int32(0))

    @pl.when(cnt > 0)
    def _():
      nch = (cnt + CHUNK - 1) // CHUNK
      pend = nch * CHUNK
      ftok = plsc.load_gather(wtok, [jnp.zeros((L,), jnp.int32)])
      fcell = plsc.load_gather(wcell, [jnp.zeros((L,), jnp.int32)])
      start = (cnt // L) * L

      def pad(t, _):
        off = start + t * L

        @pl.when(off < pend)
        def _():
          m = (off + iota) >= cnt
          wtok[pl.ds(off, L)] = jnp.where(m, ftok, wtok[pl.ds(off, L)])
          wcell[pl.ds(off, L)] = jnp.where(m, fcell, wcell[pl.ds(off, L)])
        return 0
      lax.fori_loop(0, CHUNK // L, pad, 0)

      def repack(i, _):
        r = i // (CHUNK // L)
        c = (i % (CHUNK // L)) * L
        wcell2[r, pl.ds(c, L)] = wcell[pl.ds(i * L, L)]
        return 0
      lax.fori_loop(0, nch * (CHUNK // L), repack, 0)

      def g_copy(ci, buf):
        return pltpu.make_async_copy(
            tok_b.at[wtok.at[pl.ds(ci * CHUNK, CHUNK)]], rows.at[buf], sem_g)

      def s_copy(ci, buf):
        return pltpu.make_async_copy(
            rows.at[buf], out_b.at[wcell2.at[ci]], sem_s)

      g_copy(0, 0).start()

      def move(ci, _):
        @pl.when(ci > 0)
        def _():
          s_copy(ci - 1, (ci - 1) % 2).wait()

        @pl.when(ci + 1 < nch)
        def _():
          g_copy(ci + 1, (ci + 1) % 2).start()
        g_copy(ci, ci % 2).wait()
        s_copy(ci, ci % 2).start()
        return 0
      lax.fori_loop(0, nch, move, 0)
      s_copy(nch - 1, (nch - 1) % 2).wait()

  return body(tokens, idx)


def kernel(tokens, idx, grid_size):
  del grid_size
  return _winner_scatter(tokens, idx.astype(jnp.int32))
